# baseline (speedup 1.0000x reference)
import functools
import math

import jax
import jax.numpy as jnp
import numpy as np
from jax import lax
from jax.experimental import pallas as pl
from jax.experimental.pallas import tpu as pltpu
from jax.experimental.pallas import tpu_sc as plsc

IN_DIM = 3
N_LEVELS = 16
F = 2
LOG2_T = 19
TBL = 1 << LOG2_T
MASK = TBL - 1
BASE_RES = 16
FINEST_RES = 512
N_POINTS = 262144

P1 = np.int32(np.uint32(2654435761))
P2 = np.int32(np.uint32(805459861))

NC = 2
NS = 16
NW = NC * NS

P_BLK = 512
PPW = N_POINTS // NW
NBLK = PPW // P_BLK
NV = P_BLK // 16
NQ = P_BLK // 128
SR = 8
SRW = F * SR


def _resolutions():
    b = math.exp((math.log(FINEST_RES) - math.log(BASE_RES)) / max(N_LEVELS - 1, 1))
    return [int(BASE_RES * (b ** i)) for i in range(N_LEVELS)]


def _sc_body(xt_hbm, tbl_hbm, res_hbm, out_hbm,
             xbuf, idxbuf, wbuf, cbuf, rows, outblk, resbuf,
             sem0, sem1, sem2, sem3):
    sems = (sem0, sem1, sem2, sem3)
    i1 = jnp.int32(1)
    wid = lax.axis_index("c") * jnp.int32(NS) + lax.axis_index("s")
    iota = lax.iota(jnp.int32, 16)
    zero16 = jnp.zeros((16,), jnp.int32)
    onef = jnp.float32(1.0)

    pltpu.sync_copy(res_hbm, resbuf)
    res_vec = resbuf[...]

    def block_body(blk, carry):
        gbase = wid * jnp.int32(PPW) + blk * jnp.int32(P_BLK)
        for d in range(IN_DIM):
            pltpu.sync_copy(xt_hbm.at[d, pl.ds(gbase, P_BLK)], xbuf.at[d])

        def level_body(l, carry2):
            res = jnp.max(jnp.where(iota == l, res_vec, jnp.int32(0)))
            scale = (res - i1).astype(jnp.float32)
            maxfl = res - jnp.int32(2)
            lvl_off = l * jnp.int32(TBL // SR)

            def pass_a(i, c3):
                s = i * jnp.int32(16)
                q = lax.shift_right_logical(i, jnp.int32(3))
                col = (i & jnp.int32(7)) * jnp.int32(16)
                x0 = xbuf[0, pl.ds(s, 16)] * scale
                x1 = xbuf[1, pl.ds(s, 16)] * scale
                x2 = xbuf[2, pl.ds(s, 16)] * scale
                f0 = jnp.clip(x0.astype(jnp.int32), jnp.int32(0), maxfl)
                f1 = jnp.clip(x1.astype(jnp.int32), jnp.int32(0), maxfl)
                f2 = jnp.clip(x2.astype(jnp.int32), jnp.int32(0), maxfl)
                fr0 = x0 - f0.astype(jnp.float32)
                fr1 = x1 - f1.astype(jnp.float32)
                fr2 = x2 - f2.astype(jnp.float32)
                t0 = (f0, f0 + i1)
                t1 = (f1 * P1, f1 * P1 + P1)
                t2 = (f2 * P2, f2 * P2 + P2)
                w0 = (onef - fr0, fr0)
                w1 = (onef - fr1, fr1)
                w2 = (onef - fr2, fr2)
                for c in range(8):
                    b0, b1, b2 = c & 1, (c >> 1) & 1, (c >> 2) & 1
                    h = ((t0[b0] ^ t1[b1]) ^ t2[b2]) & jnp.int32(MASK)
                    idxbuf[c, q, pl.ds(col, 16)] = (
                        lax.shift_right_logical(h, jnp.int32(3)) + lvl_off)
                    cbuf[c, pl.ds(s, 16)] = h & jnp.int32(7)
                    wbuf[c, pl.ds(s, 16)] = w0[b0] * w1[b1] * w2[b2]
                return c3

            handles = []
            for qq in range(NQ):
                lax.fori_loop(jnp.int32(8 * qq), jnp.int32(8 * qq + 8),
                              pass_a, jnp.int32(0))
                for c in range(8):
                    dst = rows.at[pl.ds((c * NQ + qq) * 128, 128), :]
                    handles.append(
                        pltpu.async_copy(tbl_hbm.at[idxbuf.at[c, qq]], dst,
                                         sems[qq]))

            def pass_b(i, c3):
                s = i * jnp.int32(16)
                prow = s + iota
                acc0 = jnp.zeros((16,), jnp.float32)
                acc1 = jnp.zeros((16,), jnp.float32)
                for c in range(8):
                    ridx = prow + jnp.int32(c * P_BLK)
                    sub = cbuf[c, pl.ds(s, 16)]
                    g0 = plsc.load_gather(rows, [ridx, sub])
                    g1 = plsc.load_gather(rows, [ridx, sub + jnp.int32(SR)])
                    w = wbuf[c, pl.ds(s, 16)]
                    acc0 = acc0 + w * g0
                    acc1 = acc1 + w * g1
                fcol = zero16 + l * jnp.int32(2)
                plsc.store_scatter(outblk, [prow, fcol], acc0)
                plsc.store_scatter(outblk, [prow, fcol + i1], acc1)
                return c3

            for qq in range(NQ):
                for c in range(8):
                    handles[qq * 8 + c].wait()
                lax.fori_loop(jnp.int32(8 * qq), jnp.int32(8 * qq + 8),
                              pass_b, jnp.int32(0))
            return carry2

        lax.fori_loop(jnp.int32(0), jnp.int32(N_LEVELS), level_body, jnp.int32(0))
        pltpu.sync_copy(outblk, out_hbm.at[pl.ds(gbase, P_BLK), :])
        return carry

    lax.fori_loop(jnp.int32(0), jnp.int32(NBLK), block_body, jnp.int32(0))


@jax.jit
def kernel(x, tables):
    with jax.enable_x64(False):
        return _run(x, tables)


def _run(x, tables):
    x = x.astype(jnp.float32)
    xt = jnp.transpose(x)
    t8 = tables.astype(jnp.float32).reshape(N_LEVELS * TBL // SR, SR, F)
    tbl = jnp.concatenate([t8[:, :, 0], t8[:, :, 1]], axis=1)
    res_arr = jnp.asarray(_resolutions(), dtype=jnp.int32)

    mesh = plsc.VectorSubcoreMesh(core_axis_name="c", subcore_axis_name="s")
    kern = pl.kernel(
        _sc_body,
        out_type=jax.ShapeDtypeStruct((x.shape[0], N_LEVELS * F), jnp.float32),
        mesh=mesh,
        compiler_params=pltpu.CompilerParams(use_tc_tiling_on_sc=False, needs_layout_passes=False),
        scratch_types=[
            pltpu.VMEM((IN_DIM, P_BLK), jnp.float32),
            pltpu.VMEM((8, NQ, 128), jnp.int32),
            pltpu.VMEM((8, P_BLK), jnp.float32),
            pltpu.VMEM((8, P_BLK), jnp.int32),
            pltpu.VMEM((8 * P_BLK, SRW), jnp.float32),
            pltpu.VMEM((P_BLK, N_LEVELS * F), jnp.float32),
            pltpu.VMEM((N_LEVELS,), jnp.int32),
            pltpu.SemaphoreType.DMA,
            pltpu.SemaphoreType.DMA,
            pltpu.SemaphoreType.DMA,
            pltpu.SemaphoreType.DMA,
        ],
    )
    return kern(xt, tbl, res_arr)


# speedup vs baseline: 1.2767x; 1.2767x over previous
import functools
import math

import jax
import jax.numpy as jnp
import numpy as np
from jax import lax
from jax.experimental import pallas as pl
from jax.experimental.pallas import tpu as pltpu
from jax.experimental.pallas import tpu_sc as plsc

IN_DIM = 3
N_LEVELS = 16
F = 2
LOG2_T = 19
TBL = 1 << LOG2_T
MASK = TBL - 1
BASE_RES = 16
FINEST_RES = 512
N_POINTS = 262144

P1 = np.int32(np.uint32(2654435761))
P2 = np.int32(np.uint32(805459861))

NC = 2
NS = 16
NW = NC * NS

P_BLK = 512
PPW = N_POINTS // NW
NBLK = PPW // P_BLK
NV = P_BLK // 16
NQ = P_BLK // 128
SR = 8
SRW = F * SR


def _resolutions():
    b = math.exp((math.log(FINEST_RES) - math.log(BASE_RES)) / max(N_LEVELS - 1, 1))
    return [int(BASE_RES * (b ** i)) for i in range(N_LEVELS)]


def _sc_body(xt_hbm, tbl_hbm, res_hbm, out_hbm,
             xbuf, idxbuf, wbuf, cbuf, rows, outblk, resbuf,
             sem0, sem1, sem2, sem3):
    sems = (sem0, sem1, sem2, sem3)
    i1 = jnp.int32(1)
    wid = lax.axis_index("c") * jnp.int32(NS) + lax.axis_index("s")
    iota = lax.iota(jnp.int32, 16)
    zero16 = jnp.zeros((16,), jnp.int32)
    onef = jnp.float32(1.0)

    pltpu.sync_copy(res_hbm, resbuf)
    res_vec = resbuf[...]

    def block_body(blk, carry):
        gbase = wid * jnp.int32(PPW) + blk * jnp.int32(P_BLK)
        for d in range(IN_DIM):
            pltpu.sync_copy(xt_hbm.at[d, pl.ds(gbase, P_BLK)], xbuf.at[d])

        def level_body(l, carry2):
            res = jnp.max(jnp.where(iota == l, res_vec, jnp.int32(0)))
            scale = (res - i1).astype(jnp.float32)
            maxfl = res - jnp.int32(2)
            lvl_off = l * jnp.int32(TBL // SR)

            def pass_a(i, c3):
                s = i * jnp.int32(16)
                q = lax.shift_right_logical(i, jnp.int32(3))
                col = (i & jnp.int32(7)) * jnp.int32(16)
                x0 = xbuf[0, pl.ds(s, 16)] * scale
                x1 = xbuf[1, pl.ds(s, 16)] * scale
                x2 = xbuf[2, pl.ds(s, 16)] * scale
                f0 = jnp.clip(x0.astype(jnp.int32), jnp.int32(0), maxfl)
                f1 = jnp.clip(x1.astype(jnp.int32), jnp.int32(0), maxfl)
                f2 = jnp.clip(x2.astype(jnp.int32), jnp.int32(0), maxfl)
                fr0 = x0 - f0.astype(jnp.float32)
                fr1 = x1 - f1.astype(jnp.float32)
                fr2 = x2 - f2.astype(jnp.float32)
                t0 = (f0, f0 + i1)
                t1 = (f1 * P1, f1 * P1 + P1)
                t2 = (f2 * P2, f2 * P2 + P2)
                w0 = (onef - fr0, fr0)
                w1 = (onef - fr1, fr1)
                w2 = (onef - fr2, fr2)
                for c in range(8):
                    b0, b1, b2 = c & 1, (c >> 1) & 1, (c >> 2) & 1
                    h = ((t0[b0] ^ t1[b1]) ^ t2[b2]) & jnp.int32(MASK)
                    idxbuf[c, q, pl.ds(col, 16)] = (
                        lax.shift_right_logical(h, jnp.int32(3)) + lvl_off)
                    cbuf[c, pl.ds(s, 16)] = h & jnp.int32(7)
                    wbuf[c, pl.ds(s, 16)] = w0[b0] * w1[b1] * w2[b2]
                return c3

            handles = []
            for qq in range(NQ):
                lax.fori_loop(jnp.int32(8 * qq), jnp.int32(8 * qq + 8),
                              pass_a, jnp.int32(0))
                for c in range(8):
                    dst = rows.at[pl.ds((c * NQ + qq) * 128, 128), :]
                    handles.append(
                        pltpu.async_copy(tbl_hbm.at[idxbuf.at[c, qq]], dst,
                                         sems[qq]))

            def pass_b(i, c3):
                s = i * jnp.int32(16)
                prow = s + iota
                acc0 = jnp.zeros((16,), jnp.float32)
                acc1 = jnp.zeros((16,), jnp.float32)
                for c in range(8):
                    ridx = prow + jnp.int32(c * P_BLK)
                    sub = cbuf[c, pl.ds(s, 16)]
                    g0 = plsc.load_gather(rows, [ridx, sub])
                    g1 = plsc.load_gather(rows, [ridx, sub + jnp.int32(SR)])
                    w = wbuf[c, pl.ds(s, 16)]
                    acc0 = acc0 + w * g0
                    acc1 = acc1 + w * g1
                frow = l * jnp.int32(2)
                outblk[frow, pl.ds(s, 16)] = acc0
                outblk[frow + i1, pl.ds(s, 16)] = acc1
                return c3

            for qq in range(NQ):
                for c in range(8):
                    handles[qq * 8 + c].wait()
                lax.fori_loop(jnp.int32(8 * qq), jnp.int32(8 * qq + 8),
                              pass_b, jnp.int32(0))
            return carry2

        lax.fori_loop(jnp.int32(0), jnp.int32(N_LEVELS), level_body, jnp.int32(0))
        for f in range(N_LEVELS * F):
            pltpu.sync_copy(outblk.at[f], out_hbm.at[f, pl.ds(gbase, P_BLK)])
        return carry

    lax.fori_loop(jnp.int32(0), jnp.int32(NBLK), block_body, jnp.int32(0))


@jax.jit
def kernel(x, tables):
    with jax.enable_x64(False):
        return _run(x, tables)


def _run(x, tables):
    x = x.astype(jnp.float32)
    xt = jnp.transpose(x)
    tbl = tables.astype(jnp.float32).reshape(N_LEVELS * TBL // SR, SR, F)
    tbl = jnp.swapaxes(tbl, 1, 2).reshape(N_LEVELS * TBL // SR, SRW)
    res_arr = jnp.asarray(_resolutions(), dtype=jnp.int32)

    mesh = plsc.VectorSubcoreMesh(core_axis_name="c", subcore_axis_name="s")
    kern = pl.kernel(
        _sc_body,
        out_type=jax.ShapeDtypeStruct((N_LEVELS * F, x.shape[0]), jnp.float32),
        mesh=mesh,
        compiler_params=pltpu.CompilerParams(use_tc_tiling_on_sc=False, needs_layout_passes=False),
        scratch_types=[
            pltpu.VMEM((IN_DIM, P_BLK), jnp.float32),
            pltpu.VMEM((8, NQ, 128), jnp.int32),
            pltpu.VMEM((8, P_BLK), jnp.float32),
            pltpu.VMEM((8, P_BLK), jnp.int32),
            pltpu.VMEM((8 * P_BLK, SRW), jnp.float32),
            pltpu.VMEM((N_LEVELS * F, P_BLK), jnp.float32),
            pltpu.VMEM((N_LEVELS,), jnp.int32),
            pltpu.SemaphoreType.DMA,
            pltpu.SemaphoreType.DMA,
            pltpu.SemaphoreType.DMA,
            pltpu.SemaphoreType.DMA,
        ],
    )
    return jnp.transpose(kern(xt, tbl, res_arr))


# speedup vs baseline: 2.1250x; 1.6644x over previous
import functools
import math

import jax
import jax.numpy as jnp
import numpy as np
from jax import lax
from jax.experimental import pallas as pl
from jax.experimental.pallas import tpu as pltpu
from jax.experimental.pallas import tpu_sc as plsc

IN_DIM = 3
N_LEVELS = 16
F = 2
LOG2_T = 19
TBL = 1 << LOG2_T
MASK = TBL - 1
BASE_RES = 16
FINEST_RES = 512
N_POINTS = 262144

P1 = np.int32(np.uint32(2654435761))
P2 = np.int32(np.uint32(805459861))

NC = 2
NS = 16
NW = NC * NS

P_BLK = 512
PPW = N_POINTS // NW
NBLK = PPW // P_BLK
NV = P_BLK // 16
NQ = P_BLK // 128
SR = 8
SRW = F * SR


def _resolutions():
    b = math.exp((math.log(FINEST_RES) - math.log(BASE_RES)) / max(N_LEVELS - 1, 1))
    return [int(BASE_RES * (b ** i)) for i in range(N_LEVELS)]


def _sc_body(xt_hbm, tbl_hbm, res_hbm, out_hbm,
             xbuf, idxbuf, wbuf, cbuf, rows, outblk, resbuf,
             sem0, sem1, sem2, sem3):
    sems = (sem0, sem1, sem2, sem3)
    i1 = jnp.int32(1)
    wid = lax.axis_index("c") * jnp.int32(NS) + lax.axis_index("s")
    iota = lax.iota(jnp.int32, 16)
    zero16 = jnp.zeros((16,), jnp.int32)
    onef = jnp.float32(1.0)

    pltpu.sync_copy(res_hbm, resbuf)
    res_vec = resbuf[...]

    def block_body(blk, carry):
        gbase = wid * jnp.int32(PPW) + blk * jnp.int32(P_BLK)
        for d in range(IN_DIM):
            pltpu.sync_copy(xt_hbm.at[d, pl.ds(gbase, P_BLK)], xbuf.at[d])

        def level_body(l, carry2):
            res = jnp.max(jnp.where(iota == l, res_vec, jnp.int32(0)))
            scale = (res - i1).astype(jnp.float32)
            maxfl = res - jnp.int32(2)
            lvl_off = l * jnp.int32(TBL // SR)

            def pass_a(i, c3):
                s = i * jnp.int32(16)
                q = lax.shift_right_logical(i, jnp.int32(3))
                col = (i & jnp.int32(7)) * jnp.int32(16)
                x0 = xbuf[0, pl.ds(s, 16)] * scale
                x1 = xbuf[1, pl.ds(s, 16)] * scale
                x2 = xbuf[2, pl.ds(s, 16)] * scale
                f0 = jnp.clip(x0.astype(jnp.int32), jnp.int32(0), maxfl)
                f1 = jnp.clip(x1.astype(jnp.int32), jnp.int32(0), maxfl)
                f2 = jnp.clip(x2.astype(jnp.int32), jnp.int32(0), maxfl)
                fr0 = x0 - f0.astype(jnp.float32)
                fr1 = x1 - f1.astype(jnp.float32)
                fr2 = x2 - f2.astype(jnp.float32)
                t0 = (f0, f0 + i1)
                t1 = (f1 * P1, f1 * P1 + P1)
                t2 = (f2 * P2, f2 * P2 + P2)
                w0 = (onef - fr0, fr0)
                w1 = (onef - fr1, fr1)
                w2 = (onef - fr2, fr2)
                for c in range(8):
                    b0, b1, b2 = c & 1, (c >> 1) & 1, (c >> 2) & 1
                    h = ((t0[b0] ^ t1[b1]) ^ t2[b2]) & jnp.int32(MASK)
                    idxbuf[c, q, pl.ds(col, 16)] = (
                        lax.shift_right_logical(h, jnp.int32(3)) + lvl_off)
                    cbuf[c, pl.ds(s, 16)] = h & jnp.int32(7)
                    wbuf[c, pl.ds(s, 16)] = w0[b0] * w1[b1] * w2[b2]
                return c3

            handles = []
            for qq in range(NQ):
                lax.fori_loop(jnp.int32(8 * qq), jnp.int32(8 * qq + 8),
                              pass_a, jnp.int32(0))
                for c in range(8):
                    dst = rows.at[pl.ds((c * NQ + qq) * 128, 128), :]
                    handles.append(
                        pltpu.async_copy(tbl_hbm.at[idxbuf.at[c, qq]], dst,
                                         sems[qq]))

            def pass_b(i, c3):
                s = i * jnp.int32(16)
                prow = s + iota
                acc0 = jnp.zeros((16,), jnp.float32)
                acc1 = jnp.zeros((16,), jnp.float32)
                for c in range(8):
                    ridx = prow + jnp.int32(c * P_BLK)
                    sub = cbuf[c, pl.ds(s, 16)]
                    g0 = plsc.load_gather(rows, [ridx, sub])
                    g1 = plsc.load_gather(rows, [ridx, sub + jnp.int32(SR)])
                    w = wbuf[c, pl.ds(s, 16)]
                    acc0 = acc0 + w * g0
                    acc1 = acc1 + w * g1
                frow = l * jnp.int32(2)
                outblk[frow, pl.ds(s, 16)] = acc0
                outblk[frow + i1, pl.ds(s, 16)] = acc1
                return c3

            for qq in range(NQ):
                for c in range(8):
                    handles[qq * 8 + c].wait()
                lax.fori_loop(jnp.int32(8 * qq), jnp.int32(8 * qq + 8),
                              pass_b, jnp.int32(0))
            return carry2

        lax.fori_loop(jnp.int32(0), jnp.int32(N_LEVELS), level_body, jnp.int32(0))
        for f in range(N_LEVELS * F):
            pltpu.sync_copy(outblk.at[f], out_hbm.at[f, pl.ds(gbase, P_BLK)])
        return carry

    lax.fori_loop(jnp.int32(0), jnp.int32(NBLK), block_body, jnp.int32(0))


@jax.jit
def kernel(x, tables):
    with jax.enable_x64(False):
        return _run(x, tables)


def _run(x, tables):
    x = x.astype(jnp.float32)
    xt = jnp.transpose(x)
    tp = jnp.transpose(tables.astype(jnp.float32), (0, 2, 1))
    t4 = tp.reshape(N_LEVELS, F, TBL // SR, SR)
    t5 = jnp.transpose(t4, (0, 2, 1, 3))
    tbl = t5.reshape(N_LEVELS * TBL // SR, SRW)
    res_arr = jnp.asarray(_resolutions(), dtype=jnp.int32)

    mesh = plsc.VectorSubcoreMesh(core_axis_name="c", subcore_axis_name="s")
    kern = pl.kernel(
        _sc_body,
        out_type=jax.ShapeDtypeStruct((N_LEVELS * F, x.shape[0]), jnp.float32),
        mesh=mesh,
        compiler_params=pltpu.CompilerParams(use_tc_tiling_on_sc=False, needs_layout_passes=False),
        scratch_types=[
            pltpu.VMEM((IN_DIM, P_BLK), jnp.float32),
            pltpu.VMEM((8, NQ, 128), jnp.int32),
            pltpu.VMEM((8, P_BLK), jnp.float32),
            pltpu.VMEM((8, P_BLK), jnp.int32),
            pltpu.VMEM((8 * P_BLK, SRW), jnp.float32),
            pltpu.VMEM((N_LEVELS * F, P_BLK), jnp.float32),
            pltpu.VMEM((N_LEVELS,), jnp.int32),
            pltpu.SemaphoreType.DMA,
            pltpu.SemaphoreType.DMA,
            pltpu.SemaphoreType.DMA,
            pltpu.SemaphoreType.DMA,
        ],
    )
    return jnp.transpose(kern(xt, tbl, res_arr))


# speedup vs baseline: 2.1269x; 1.0009x over previous
import functools
import math

import jax
import jax.numpy as jnp
import numpy as np
from jax import lax
from jax.experimental import pallas as pl
from jax.experimental.pallas import tpu as pltpu
from jax.experimental.pallas import tpu_sc as plsc

IN_DIM = 3
N_LEVELS = 16
F = 2
LOG2_T = 19
TBL = 1 << LOG2_T
MASK = TBL - 1
BASE_RES = 16
FINEST_RES = 512
N_POINTS = 262144

P1 = np.int32(np.uint32(2654435761))
P2 = np.int32(np.uint32(805459861))

NC = 2
NS = 16
NW = NC * NS

P_BLK = 512
PPW = N_POINTS // NW
NBLK = PPW // P_BLK
NV = P_BLK // 16
NQ = P_BLK // 128
SR = 8
SRW = F * SR


def _resolutions():
    b = math.exp((math.log(FINEST_RES) - math.log(BASE_RES)) / max(N_LEVELS - 1, 1))
    return [int(BASE_RES * (b ** i)) for i in range(N_LEVELS)]


def _sc_body(xt_hbm, tbl_hbm, res_hbm, out_hbm,
             xbuf, idxbuf, wbuf, cbuf, rows, outblk, resbuf,
             sem0, sem1, sem2, sem3):
    sems = (sem0, sem1, sem2, sem3)
    i1 = jnp.int32(1)
    wid = lax.axis_index("c") * jnp.int32(NS) + lax.axis_index("s")
    iota = lax.iota(jnp.int32, 16)
    zero16 = jnp.zeros((16,), jnp.int32)
    onef = jnp.float32(1.0)

    pltpu.sync_copy(res_hbm, resbuf)
    res_vec = resbuf[...]

    def block_body(blk, carry):
        gbase = wid * jnp.int32(PPW) + blk * jnp.int32(P_BLK)
        for d in range(IN_DIM):
            pltpu.sync_copy(xt_hbm.at[d, pl.ds(gbase, P_BLK)], xbuf.at[d])

        def level_body(l, carry2):
            res = jnp.max(jnp.where(iota == l, res_vec, jnp.int32(0)))
            scale = (res - i1).astype(jnp.float32)
            maxfl = res - jnp.int32(2)
            lvl_off = l * jnp.int32(TBL // SR)

            def pass_a(i, c3):
                s = i * jnp.int32(16)
                q = lax.shift_right_logical(i, jnp.int32(3))
                col = (i & jnp.int32(7)) * jnp.int32(16)
                x0 = xbuf[0, pl.ds(s, 16)] * scale
                x1 = xbuf[1, pl.ds(s, 16)] * scale
                x2 = xbuf[2, pl.ds(s, 16)] * scale
                f0 = jnp.clip(x0.astype(jnp.int32), jnp.int32(0), maxfl)
                f1 = jnp.clip(x1.astype(jnp.int32), jnp.int32(0), maxfl)
                f2 = jnp.clip(x2.astype(jnp.int32), jnp.int32(0), maxfl)
                fr0 = x0 - f0.astype(jnp.float32)
                fr1 = x1 - f1.astype(jnp.float32)
                fr2 = x2 - f2.astype(jnp.float32)
                t0 = (f0, f0 + i1)
                t1 = (f1 * P1, f1 * P1 + P1)
                t2 = (f2 * P2, f2 * P2 + P2)
                w0 = (onef - fr0, fr0)
                w1 = (onef - fr1, fr1)
                w2 = (onef - fr2, fr2)
                for c in range(8):
                    b0, b1, b2 = c & 1, (c >> 1) & 1, (c >> 2) & 1
                    h = ((t0[b0] ^ t1[b1]) ^ t2[b2]) & jnp.int32(MASK)
                    idxbuf[c, q, pl.ds(col, 16)] = (
                        lax.shift_right_logical(h, jnp.int32(3)) + lvl_off)
                    cbuf[c, pl.ds(s, 16)] = h & jnp.int32(7)
                    wbuf[c, pl.ds(s, 16)] = w0[b0] * w1[b1] * w2[b2]
                return c3

            handles = []
            for qq in range(NQ):
                lax.fori_loop(jnp.int32(8 * qq), jnp.int32(8 * qq + 8),
                              pass_a, jnp.int32(0))
                for c in range(8):
                    dst = rows.at[pl.ds((c * NQ + qq) * 128, 128), :]
                    handles.append(
                        pltpu.async_copy(tbl_hbm.at[idxbuf.at[c, qq]], dst,
                                         sems[qq]))

            def pass_b(i, c3):
                s = i * jnp.int32(16)
                prow = s + iota
                acc0 = jnp.zeros((16,), jnp.float32)
                acc1 = jnp.zeros((16,), jnp.float32)
                for c in range(8):
                    ridx = prow + jnp.int32(c * P_BLK)
                    sub = cbuf[c, pl.ds(s, 16)]
                    g0 = plsc.load_gather(rows, [ridx, sub])
                    g1 = plsc.load_gather(rows, [ridx, sub + jnp.int32(SR)])
                    w = wbuf[c, pl.ds(s, 16)]
                    acc0 = acc0 + w * g0
                    acc1 = acc1 + w * g1
                frow = l * jnp.int32(2)
                outblk[frow, pl.ds(s, 16)] = acc0
                outblk[frow + i1, pl.ds(s, 16)] = acc1
                return c3

            for qq in range(NQ):
                for c in range(8):
                    handles[qq * 8 + c].wait()
                lax.fori_loop(jnp.int32(8 * qq), jnp.int32(8 * qq + 8),
                              pass_b, jnp.int32(0))
            return carry2

        lax.fori_loop(jnp.int32(0), jnp.int32(N_LEVELS), level_body, jnp.int32(0))
        for f in range(N_LEVELS * F):
            pltpu.sync_copy(outblk.at[f], out_hbm.at[f, pl.ds(gbase, P_BLK)])
        return carry

    lax.fori_loop(jnp.int32(0), jnp.int32(NBLK), block_body, jnp.int32(0))


@jax.jit
def kernel(x, tables):
    with jax.enable_x64(False):
        return _run(x, tables)


def _run(x, tables):
    x = x.astype(jnp.float32)
    xt = jnp.transpose(x)
    t4 = tables.astype(jnp.float32).reshape(N_LEVELS, TBL // SR, SR, F)
    t5 = jnp.transpose(t4, (0, 1, 3, 2))
    tbl = t5.reshape(N_LEVELS * TBL // SR, SRW)
    res_arr = jnp.asarray(_resolutions(), dtype=jnp.int32)

    mesh = plsc.VectorSubcoreMesh(core_axis_name="c", subcore_axis_name="s")
    kern = pl.kernel(
        _sc_body,
        out_type=jax.ShapeDtypeStruct((N_LEVELS * F, x.shape[0]), jnp.float32),
        mesh=mesh,
        compiler_params=pltpu.CompilerParams(use_tc_tiling_on_sc=False, needs_layout_passes=False),
        scratch_types=[
            pltpu.VMEM((IN_DIM, P_BLK), jnp.float32),
            pltpu.VMEM((8, NQ, 128), jnp.int32),
            pltpu.VMEM((8, P_BLK), jnp.float32),
            pltpu.VMEM((8, P_BLK), jnp.int32),
            pltpu.VMEM((8 * P_BLK, SRW), jnp.float32),
            pltpu.VMEM((N_LEVELS * F, P_BLK), jnp.float32),
            pltpu.VMEM((N_LEVELS,), jnp.int32),
            pltpu.SemaphoreType.DMA,
            pltpu.SemaphoreType.DMA,
            pltpu.SemaphoreType.DMA,
            pltpu.SemaphoreType.DMA,
        ],
    )
    return jnp.transpose(kern(xt, tbl, res_arr))


# speedup vs baseline: 2.1350x; 1.0038x over previous
import functools
import math

import jax
import jax.numpy as jnp
import numpy as np
from jax import lax
from jax.experimental import pallas as pl
from jax.experimental.pallas import tpu as pltpu
from jax.experimental.pallas import tpu_sc as plsc

IN_DIM = 3
N_LEVELS = 16
F = 2
LOG2_T = 19
TBL = 1 << LOG2_T
MASK = TBL - 1
BASE_RES = 16
FINEST_RES = 512
N_POINTS = 262144

P1 = np.int32(np.uint32(2654435761))
P2 = np.int32(np.uint32(805459861))

NC = 2
NS = 16
NW = NC * NS

P_BLK = 512
PPW = N_POINTS // NW
NBLK = PPW // P_BLK
NV = P_BLK // 16
NQ = P_BLK // 128
SR = 8
SRW = F * SR


def _resolutions():
    b = math.exp((math.log(FINEST_RES) - math.log(BASE_RES)) / max(N_LEVELS - 1, 1))
    return [int(BASE_RES * (b ** i)) for i in range(N_LEVELS)]


def _sc_body(xt_hbm, tbl_hbm, res_hbm, out_hbm,
             xbuf, idxbuf, wbuf, cbuf, rows, outblk, resbuf,
             sem0, sem1, sem2, sem3):
    sems = (sem0, sem1, sem2, sem3)
    i1 = jnp.int32(1)
    wid = lax.axis_index("c") * jnp.int32(NS) + lax.axis_index("s")
    iota = lax.iota(jnp.int32, 16)
    zero16 = jnp.zeros((16,), jnp.int32)
    onef = jnp.float32(1.0)

    pltpu.sync_copy(res_hbm, resbuf)
    res_vec = resbuf[...]

    def block_body(blk, carry):
        gbase = wid * jnp.int32(PPW) + blk * jnp.int32(P_BLK)
        for d in range(IN_DIM):
            pltpu.sync_copy(xt_hbm.at[d, pl.ds(gbase, P_BLK)], xbuf.at[d])

        def level_body(l, carry2):
            res = jnp.max(jnp.where(iota == l, res_vec, jnp.int32(0)))
            scale = (res - i1).astype(jnp.float32)
            maxfl = res - jnp.int32(2)
            lvl_off = l * jnp.int32(TBL // SR)

            def pass_a(i, c3):
                s = i * jnp.int32(16)
                q = lax.shift_right_logical(i, jnp.int32(3))
                col = (i & jnp.int32(7)) * jnp.int32(16)
                x0 = xbuf[0, pl.ds(s, 16)] * scale
                x1 = xbuf[1, pl.ds(s, 16)] * scale
                x2 = xbuf[2, pl.ds(s, 16)] * scale
                f0 = jnp.clip(x0.astype(jnp.int32), jnp.int32(0), maxfl)
                f1 = jnp.clip(x1.astype(jnp.int32), jnp.int32(0), maxfl)
                f2 = jnp.clip(x2.astype(jnp.int32), jnp.int32(0), maxfl)
                fr0 = x0 - f0.astype(jnp.float32)
                fr1 = x1 - f1.astype(jnp.float32)
                fr2 = x2 - f2.astype(jnp.float32)
                t0 = (f0, f0 + i1)
                t1 = (f1 * P1, f1 * P1 + P1)
                t2 = (f2 * P2, f2 * P2 + P2)
                w0 = (onef - fr0, fr0)
                w1 = (onef - fr1, fr1)
                w2 = (onef - fr2, fr2)
                for c in range(8):
                    b0, b1, b2 = c & 1, (c >> 1) & 1, (c >> 2) & 1
                    h = ((t0[b0] ^ t1[b1]) ^ t2[b2]) & jnp.int32(MASK)
                    idxbuf[c, q, pl.ds(col, 16)] = (
                        lax.shift_right_logical(h, jnp.int32(3)) + lvl_off)
                    cbuf[c, pl.ds(s, 16)] = h & jnp.int32(7)
                    wbuf[c, pl.ds(s, 16)] = w0[b0] * w1[b1] * w2[b2]
                return c3

            def fire(c, qq, sem):
                dst = rows.at[pl.ds((c * jnp.int32(NQ) + qq) * jnp.int32(128),
                                    128), :]
                return pltpu.async_copy(tbl_hbm.at[idxbuf.at[c, qq]], dst, sem)

            for qq in range(NQ):
                lax.fori_loop(jnp.int32(8 * qq), jnp.int32(8 * qq + 8),
                              pass_a, jnp.int32(0))

                def fire_c(c, c3, _qq=qq):
                    fire(c, jnp.int32(_qq), sems[_qq])
                    return c3

                lax.fori_loop(jnp.int32(0), jnp.int32(8), fire_c, jnp.int32(0))

            def pass_b(i, c3):
                s = i * jnp.int32(16)
                prow = s + iota
                acc0 = jnp.zeros((16,), jnp.float32)
                acc1 = jnp.zeros((16,), jnp.float32)
                for c in range(8):
                    ridx = prow + jnp.int32(c * P_BLK)
                    sub = cbuf[c, pl.ds(s, 16)]
                    g0 = plsc.load_gather(rows, [ridx, sub])
                    g1 = plsc.load_gather(rows, [ridx, sub + jnp.int32(SR)])
                    w = wbuf[c, pl.ds(s, 16)]
                    acc0 = acc0 + w * g0
                    acc1 = acc1 + w * g1
                frow = l * jnp.int32(2)
                outblk[frow, pl.ds(s, 16)] = acc0
                outblk[frow + i1, pl.ds(s, 16)] = acc1
                return c3

            for qq in range(NQ):
                def wait_c(c, c3, _qq=qq):
                    dst = rows.at[pl.ds(
                        (c * jnp.int32(NQ) + jnp.int32(_qq)) * jnp.int32(128),
                        128), :]
                    pltpu.make_async_copy(
                        tbl_hbm.at[idxbuf.at[c, jnp.int32(_qq)]], dst,
                        sems[_qq]).wait()
                    return c3

                lax.fori_loop(jnp.int32(0), jnp.int32(8), wait_c, jnp.int32(0))
                lax.fori_loop(jnp.int32(8 * qq), jnp.int32(8 * qq + 8),
                              pass_b, jnp.int32(0))
            return carry2

        lax.fori_loop(jnp.int32(0), jnp.int32(N_LEVELS), level_body, jnp.int32(0))
        for f in range(N_LEVELS * F):
            pltpu.sync_copy(outblk.at[f], out_hbm.at[f, pl.ds(gbase, P_BLK)])
        return carry

    lax.fori_loop(jnp.int32(0), jnp.int32(NBLK), block_body, jnp.int32(0))


@jax.jit
def kernel(x, tables):
    with jax.enable_x64(False):
        return _run(x, tables)


def _run(x, tables):
    x = x.astype(jnp.float32)
    xt = jnp.transpose(x)
    t4 = tables.astype(jnp.float32).reshape(N_LEVELS, TBL // SR, SR, F)
    t5 = jnp.transpose(t4, (0, 1, 3, 2))
    tbl = t5.reshape(N_LEVELS * TBL // SR, SRW)
    res_arr = jnp.asarray(_resolutions(), dtype=jnp.int32)

    mesh = plsc.VectorSubcoreMesh(core_axis_name="c", subcore_axis_name="s")
    kern = pl.kernel(
        _sc_body,
        out_type=jax.ShapeDtypeStruct((N_LEVELS * F, x.shape[0]), jnp.float32),
        mesh=mesh,
        compiler_params=pltpu.CompilerParams(use_tc_tiling_on_sc=False, needs_layout_passes=False),
        scratch_types=[
            pltpu.VMEM((IN_DIM, P_BLK), jnp.float32),
            pltpu.VMEM((8, NQ, 128), jnp.int32),
            pltpu.VMEM((8, P_BLK), jnp.float32),
            pltpu.VMEM((8, P_BLK), jnp.int32),
            pltpu.VMEM((8 * P_BLK, SRW), jnp.float32),
            pltpu.VMEM((N_LEVELS * F, P_BLK), jnp.float32),
            pltpu.VMEM((N_LEVELS,), jnp.int32),
            pltpu.SemaphoreType.DMA,
            pltpu.SemaphoreType.DMA,
            pltpu.SemaphoreType.DMA,
            pltpu.SemaphoreType.DMA,
        ],
    )
    return jnp.transpose(kern(xt, tbl, res_arr))
